# baseline (speedup 1.0000x reference)
import functools

import jax
import jax.numpy as jnp
from jax import lax
from jax.experimental import pallas as pl
from jax.experimental.pallas import tpu as pltpu
from jax.experimental.pallas import tpu_sc as plsc

N = 10000
E = 320000
A = 128
NCRYS = 200
NW = 32
PER_W = E // NW
CH = 80
NCHUNK = PER_W // CH
NR_CH = 80
NR_NCHUNK = N // NR_CH

def _lrelu(x):
    return jnp.where(x >= 0, x, 0.2 * x)


def _sc_mesh():
    return plsc.VectorSubcoreMesh(core_axis_name="c", subcore_axis_name="s")


def _sc_gather_pair_body(xa_hbm, xb_hbm, i1_hbm, i2_hbm, g1_hbm, g2_hbm,
                         i1v, i2v, r1, r2, sem1, sem2):
    wid = lax.axis_index("s") * 2 + lax.axis_index("c")
    base = wid * PER_W

    def body(c, carry):
        off = base + c * CH
        pltpu.sync_copy(i1_hbm.at[pl.ds(off, CH)], i1v)
        pltpu.sync_copy(i2_hbm.at[pl.ds(off, CH)], i2v)
        cp1 = pltpu.async_copy(xa_hbm.at[i1v], r1, sem1)
        cp2 = pltpu.async_copy(xb_hbm.at[i2v], r2, sem2)
        cp1.wait()
        cp2.wait()
        pltpu.sync_copy(r1, g1_hbm.at[pl.ds(off, CH)])
        pltpu.sync_copy(r2, g2_hbm.at[pl.ds(off, CH)])
        return carry

    lax.fori_loop(0, NCHUNK, body, 0)


@functools.cache
def _sc_gather_pair_call():
    return pl.kernel(
        _sc_gather_pair_body,
        out_type=(jax.ShapeDtypeStruct((E, A), jnp.float32),
                  jax.ShapeDtypeStruct((E, A), jnp.float32)),
        mesh=_sc_mesh(),
        scratch_types=[
            pltpu.VMEM((CH,), jnp.int32),
            pltpu.VMEM((CH,), jnp.int32),
            pltpu.VMEM((CH, A), jnp.float32),
            pltpu.VMEM((CH, A), jnp.float32),
            pltpu.SemaphoreType.DMA,
            pltpu.SemaphoreType.DMA,
        ],
    )


def _sc_gather_pair(xa, xb, i1, i2):
    return _sc_gather_pair_call()(xa, xb, i1, i2)


def _sc_scatter_add_body(rows_hbm, i1_hbm, zeros_hbm, out_hbm, iv, rows, acc):
    cid = lax.axis_index("c")
    sid = lax.axis_index("s")
    wid = sid * 2 + cid
    init_iters = (NR_NCHUNK + 15) // 16

    def init(t, carry):
        c = sid + t * 16

        @pl.when(c < NR_NCHUNK)
        def _():
            pltpu.sync_copy(zeros_hbm.at[pl.ds(c * NR_CH, NR_CH)], rows)
            pltpu.sync_copy(rows, acc.at[pl.ds(c * NR_CH, NR_CH)])

        return carry

    lax.fori_loop(0, init_iters, init, 0)
    plsc.subcore_barrier()

    def body(c, carry):
        off = wid * PER_W + c * CH
        pltpu.sync_copy(i1_hbm.at[pl.ds(off, CH)], iv)
        pltpu.sync_copy(rows_hbm.at[pl.ds(off, CH)], rows)
        pltpu.sync_copy(rows, acc.at[iv], add=True)
        return carry

    lax.fori_loop(0, NCHUNK, body, 0)
    plsc.subcore_barrier()

    def drain(t, carry):
        c = sid + t * 16

        @pl.when(c < NR_NCHUNK)
        def _():
            pltpu.sync_copy(acc.at[pl.ds(c * NR_CH, NR_CH)], rows)
            pltpu.sync_copy(rows, out_hbm.at[cid, pl.ds(c * NR_CH, NR_CH)])

        return carry

    lax.fori_loop(0, init_iters, drain, 0)


@functools.cache
def _sc_scatter_add_call():
    return pl.kernel(
        _sc_scatter_add_body,
        out_type=jax.ShapeDtypeStruct((2, N, A), jnp.float32),
        mesh=_sc_mesh(),
        scratch_types=[
            pltpu.VMEM((CH,), jnp.int32),
            pltpu.VMEM((CH, A), jnp.float32),
            pltpu.VMEM_SHARED((N, A), jnp.float32),
        ],
    )


def _sc_scatter_add(rows, i1, zeros):
    return _sc_scatter_add_call()(rows, i1, zeros)


_SF_CH = 80
_SF_NCHUNK = N // _SF_CH


def _sc_gather_sf_body(tab_hbm, ai_hbm, out_hbm, iv, rows, sem):
    wid = lax.axis_index("s") * 2 + lax.axis_index("c")
    iters = (_SF_NCHUNK + NW - 1) // NW

    def body(t, carry):
        c = wid + t * NW

        @pl.when(c < _SF_NCHUNK)
        def _():
            off = c * _SF_CH
            pltpu.sync_copy(ai_hbm.at[pl.ds(off, _SF_CH)], iv)
            pltpu.async_copy(tab_hbm.at[iv], rows, sem).wait()
            pltpu.sync_copy(rows, out_hbm.at[pl.ds(off, _SF_CH)])

        return carry

    lax.fori_loop(0, iters, body, 0)


@functools.cache
def _sc_gather_sf_call():
    return pl.kernel(
        _sc_gather_sf_body,
        out_type=jax.ShapeDtypeStruct((N, 512), jnp.float32),
        mesh=_sc_mesh(),
        scratch_types=[
            pltpu.VMEM((_SF_CH,), jnp.int32),
            pltpu.VMEM((_SF_CH, 512), jnp.float32),
            pltpu.SemaphoreType.DMA,
        ],
    )


def _sc_gather_sf(tab, ai):
    return _sc_gather_sf_call()(tab, ai)


def _dot(a, b):
    return jnp.dot(a, b, preferred_element_type=jnp.float32)


def _emb_pre_body(af, wn, bn, wa, wb, x_o, xa_o, xb_o):
    x = _dot(af[...], wn[...]) + bn[...]
    x_o[...] = x
    xa_o[...] = _dot(x, wa[...])
    xb_o[...] = _dot(x, wb[...])


def _emb_pre(atom_fea, wn_t, bn, wa_t, wb_t):
    f = jax.ShapeDtypeStruct
    return pl.pallas_call(
        _emb_pre_body,
        out_shape=(f((N, A), jnp.float32), f((N, A), jnp.float32),
                   f((N, A), jnp.float32)),
    )(atom_fea, wn_t, bn, wa_t, wb_t)


EB = 1600
EGRID = E // EB


def _edge_mlp_body(has_emb, src, g1, g2, *refs):
    if has_emb:
        we, be = refs[0], refs[1]
        refs = refs[2:]
    w1c, b1, w2, b2, w3, b3, ek_o, enew_o = refs
    if has_emb:
        e = _dot(src[...], we[...]) + be[...]
    else:
        e = src[...]
    t = _lrelu(_dot(e, w1c[...]) + g1[...] + g2[...] + b1[...])
    t = _lrelu(_dot(t, w2[...]) + b2[...])
    ek = _dot(t, w3[...]) + b3[...]
    ek_o[...] = ek
    enew_o[...] = e + ek


def _edge_mlp(src, g1, g2, weights, has_emb):
    f = jax.ShapeDtypeStruct
    kin = src.shape[1]
    full = lambda s: pl.BlockSpec(s, lambda i: (0, 0))
    row = lambda s: pl.BlockSpec(s, lambda i: (i, 0))
    in_specs = [row((EB, kin)), row((EB, A)), row((EB, A))]
    in_specs += [full(w.shape) for w in weights]
    return pl.pallas_call(
        functools.partial(_edge_mlp_body, has_emb),
        grid=(EGRID,),
        in_specs=in_specs,
        out_specs=(row((EB, A)), row((EB, A))),
        out_shape=(f((E, A), jnp.float32), f((E, A), jnp.float32)),
    )(src, g1, g2, *weights)


def _node_mlp_body(x, rho_p, invn, wv1a, bv1, wv1b, wv2, bv2, wv3, bv3,
                   bng, bnb, xn_o):
    rho = (rho_p[0] + rho_p[1]) * invn[...]
    t = _lrelu(_dot(x[...], wv1a[...]) + _dot(rho, wv1b[...]) + bv1[...])
    t = _lrelu(_dot(t, wv2[...]) + bv2[...])
    vi = _dot(t, wv3[...]) + bv3[...]
    m = jnp.mean(vi, axis=0, keepdims=True)
    d = vi - m
    v = jnp.mean(d * d, axis=0, keepdims=True)
    vi = bng[...] * d / jnp.sqrt(v + 1e-5) + bnb[...]
    xn_o[...] = x[...] + vi


def _node_mlp(x, rho_p, invn, wv):
    return pl.pallas_call(
        _node_mlp_body,
        out_shape=jax.ShapeDtypeStruct((N, A), jnp.float32),
    )(x, rho_p, invn, *wv)


NPB = 2000


def _node_pre_body(x, wa, wb, xa_o, xb_o):
    xa_o[...] = _dot(x[...], wa[...])
    xb_o[...] = _dot(x[...], wb[...])


def _node_pre(x, wa_t, wb_t):
    f = jax.ShapeDtypeStruct
    full = lambda s: pl.BlockSpec(s, lambda i: (0, 0))
    row = lambda s: pl.BlockSpec(s, lambda i: (i, 0))
    return pl.pallas_call(
        _node_pre_body,
        grid=(N // NPB,),
        in_specs=[row((NPB, A)), full((A, A)), full((A, A))],
        out_specs=(row((NPB, A)), row((NPB, A))),
        out_shape=(f((N, A), jnp.float32), f((N, A), jnp.float32)),
    )(x, wa_t, wb_t)


def _sf_table_body(sf, wf, bf, o):
    o[...] = _dot(sf[...], wf[...]) + bf[...]


def _sf_table(structure_feature, wfeat_t, bfeat):
    return pl.pallas_call(
        _sf_table_body,
        out_shape=jax.ShapeDtypeStruct((NCRYS, 512), jnp.float32),
    )(structure_feature, wfeat_t, bfeat)


ZB = 2000


def _charge_pre_body(x, eks_p, invn, sf2g, wfa, wfb, banf, wpa, wpb, bpos,
                     z_o, st_o):
    eks = (eks_p[0] + eks_p[1]) * invn[...]
    anf_lin = _dot(x[...], wfa[...]) + _dot(eks, wfb[...]) + banf[...]
    z = (_dot(anf_lin, wpa[...]) + _dot(sf2g[...], wpb[...]) + bpos[...])
    z_o[...] = z

    @pl.when(pl.program_id(0) == 0)
    def _():
        st_o[...] = jnp.zeros_like(st_o)

    st_o[...] += jnp.sum(z, axis=0, keepdims=True)


def _charge_pre(x, eks_p, invn, sf2g, wfa_t, wfb_t, banf, wpa_t, wpb_t, bpos):
    f = jax.ShapeDtypeStruct
    full = lambda s: pl.BlockSpec(s, lambda i: tuple(0 for _ in s))
    row = lambda s: pl.BlockSpec(s, lambda i: (i,) + tuple(0 for _ in s[1:]))
    return pl.pallas_call(
        _charge_pre_body,
        grid=(N // ZB,),
        in_specs=[row((ZB, A)), pl.BlockSpec((2, ZB, A), lambda i: (0, i, 0)),
                  row((ZB, 1)), row((ZB, 512)),
                  full((A, A)), full((A, A)), full((1, A)),
                  full((A, 512)), full((512, 512)), full((1, 512))],
        out_specs=(row((ZB, 512)), full((1, 512))),
        out_shape=(f((N, 512), jnp.float32), f((1, 512), jnp.float32)),
    )(x, eks_p, invn, sf2g, wfa_t, wfb_t, banf, wpa_t, wpb_t, bpos)


VB = 2000


def _colvar_body(nslice, a, m, st_o):
    @pl.when(pl.program_id(0) == 0)
    def _():
        st_o[...] = jnp.zeros_like(st_o)

    c = m.shape[1]
    acc = jnp.zeros((1, c), jnp.float32)
    for p in range(nslice):
        d = a[:, p * c:(p + 1) * c] - m[...]
        acc += jnp.sum(d * d, axis=0, keepdims=True)
    st_o[...] += acc


def _colvar(a, mean):
    nslice = a.shape[1] // mean.shape[1]
    c = mean.shape[1]
    full = lambda s: pl.BlockSpec(s, lambda i: (0, 0))
    row = lambda s: pl.BlockSpec(s, lambda i: (i, 0))
    return pl.pallas_call(
        functools.partial(_colvar_body, nslice),
        grid=(N // VB,),
        in_specs=[row((VB, a.shape[1])), full((1, c))],
        out_specs=full((1, c)),
        out_shape=jax.ShapeDtypeStruct((1, c), jnp.float32),
    )(a, mean)


CB1 = 1000
CB2 = 400


def _conv1_body(z, scale, offs, w1f, b1c, h1_o, st_o):
    c = _lrelu(z[...] * scale[...] + offs[...])

    @pl.when(pl.program_id(0) == 0)
    def _():
        st_o[...] = jnp.zeros_like(st_o)

    s = jnp.zeros((1, 512), jnp.float32)
    for p in range(6):
        hp = _dot(c[:, p * 64:(p + 3) * 64], w1f[...]) + b1c[...]
        h1_o[:, p * 512:(p + 1) * 512] = hp
        s += jnp.sum(hp, axis=0, keepdims=True)
    st_o[...] += s


def _conv1(z, scale, offs, w1f, b1c):
    f = jax.ShapeDtypeStruct
    full = lambda s: pl.BlockSpec(s, lambda i: tuple(0 for _ in s))
    row = lambda s: pl.BlockSpec(s, lambda i: (i,) + tuple(0 for _ in s[1:]))
    return pl.pallas_call(
        _conv1_body,
        grid=(N // CB1,),
        in_specs=[row((CB1, 512)), full((1, 512)), full((1, 512)),
                  full((192, 512)), full((1, 512))],
        out_specs=(row((CB1, 3072)), full((1, 512))),
        out_shape=(f((N, 3072), jnp.float32), f((1, 512), jnp.float32)),
    )(z, scale, offs, w1f, b1c)


def _conv2_body(h1, scale, offs, w2f, b2c, h2_o, st_o):
    a = _lrelu(h1[...] * scale[...] + offs[...])

    @pl.when(pl.program_id(0) == 0)
    def _():
        st_o[...] = jnp.zeros_like(st_o)

    s = jnp.zeros((1, 512), jnp.float32)
    for q in range(4):
        hq = _dot(a[:, q * 512:(q + 3) * 512], w2f[...]) + b2c[...]
        h2_o[:, q * 512:(q + 1) * 512] = hq
        s += jnp.sum(hq, axis=0, keepdims=True)
    st_o[...] += s


def _conv2(h1, scale_t, offs_t, w2f, b2c):
    f = jax.ShapeDtypeStruct
    full = lambda s: pl.BlockSpec(s, lambda i: tuple(0 for _ in s))
    row = lambda s: pl.BlockSpec(s, lambda i: (i,) + tuple(0 for _ in s[1:]))
    return pl.pallas_call(
        _conv2_body,
        grid=(N // CB2,),
        in_specs=[row((CB2, 3072)), full((1, 3072)), full((1, 3072)),
                  full((1536, 512)), full((1, 512))],
        out_specs=(row((CB2, 2048)), full((1, 512))),
        out_shape=(f((N, 2048), jnp.float32), f((1, 512), jnp.float32)),
    )(h1, scale_t, offs_t, w2f, b2c)


def _conv345_body(h2, scale, offs, w3f, b3c, w4f, b4c, w5p, b5p, out_o):
    a = _lrelu(h2[...] * scale[...] + offs[...])
    h3 = []
    for q in range(4):
        if q == 0:
            hq = _dot(a[:, 0:1024], w3f[512:1536, :])
        elif q == 3:
            hq = _dot(a[:, 1024:2048], w3f[0:1024, :])
        else:
            hq = _dot(a[:, (q - 1) * 512:(q + 2) * 512], w3f[...])
        h3.append(_lrelu(hq + b3c[...]))
    h3 = jnp.concatenate(h3, axis=1)
    h4 = []
    for q in range(4):
        if q == 0:
            hq = _dot(h3[:, 0:512], w4f[256:768, :])
        elif q == 3:
            hq = _dot(h3[:, 512:1024], w4f[0:512, :])
        else:
            hq = _dot(h3[:, (q - 1) * 256:(q + 2) * 256], w4f[...])
        h4.append(_lrelu(hq + b4c[...]))
    h4 = jnp.concatenate(h4, axis=1)
    out_o[...] = _dot(h4, w5p[...]) + b5p[...]


def _conv345(h2, scale_t, offs_t, w3f, b3c, w4f, b4c, w5p, b5p):
    full = lambda s: pl.BlockSpec(s, lambda i: tuple(0 for _ in s))
    row = lambda s: pl.BlockSpec(s, lambda i: (i,) + tuple(0 for _ in s[1:]))
    return pl.pallas_call(
        _conv345_body,
        grid=(N // CB2,),
        in_specs=[row((CB2, 2048)), full((1, 2048)), full((1, 2048)),
                  full((1536, 256)), full((1, 256)),
                  full((768, 256)), full((1, 256)),
                  full((1024, 8)), full((1, 8))],
        out_specs=row((CB2, 8)),
        out_shape=jax.ShapeDtypeStruct((N, 8), jnp.float32),
    )(h2, scale_t, offs_t, w3f, b3c, w4f, b4c, w5p, b5p)


def _bn_affine(a, ssum, count, g, b):
    m = ssum / count
    ss = _colvar(a, m.reshape(1, -1))
    v = ss[0] / count
    scale = g / jnp.sqrt(v + 1e-5)
    return scale, b - m * scale


def kernel(atom_fea, nbr_fea, nbr_fea_idx1, nbr_fea_idx2, num_nbrs, atom_idx,
           structure_feature, params):
    p = params
    f32 = jnp.float32
    idx1 = nbr_fea_idx1.astype(jnp.int32)
    idx2 = nbr_fea_idx2.astype(jnp.int32)
    ai = atom_idx.astype(jnp.int32)
    invn = (1.0 / num_nbrs).reshape(N, 1).astype(f32)
    zeros = jnp.zeros((N, A), f32)

    wn_t = p['node_emb'][0].T
    bn_ = p['node_emb'][1].reshape(1, A)
    we_t = p['edge_emb'][0].T
    be_ = p['edge_emb'][1].reshape(1, A)

    convs = []
    for c in p['convs']:
        w1 = c['phi_e'][0][0]
        convs.append(dict(
            wa_t=w1[:, 0:A].T, wb_t=w1[:, A:2 * A].T, w1c_t=w1[:, 2 * A:].T,
            b1=c['phi_e'][0][1].reshape(1, A),
            w2_t=c['phi_e'][1][0].T, b2=c['phi_e'][1][1].reshape(1, A),
            w3_t=c['phi_e'][2][0].T, b3=c['phi_e'][2][1].reshape(1, A),
            wv1a_t=c['phi_v'][0][0][:, 0:A].T,
            wv1b_t=c['phi_v'][0][0][:, A:2 * A].T,
            bv1=c['phi_v'][0][1].reshape(1, A),
            wv2_t=c['phi_v'][1][0].T, bv2=c['phi_v'][1][1].reshape(1, A),
            wv3_t=c['phi_v'][2][0].T, bv3=c['phi_v'][2][1].reshape(1, A),
            bng=c['bn_g'].reshape(1, A), bnb=c['bn_b'].reshape(1, A),
        ))

    hh, cc = jnp.meshgrid(jnp.arange(8), jnp.arange(64), indexing='ij')
    perm = (cc * 8 + hh).reshape(512)
    wanf, banf = p['anf_emb']
    wpos, bpos = p['phi_pos']['lin']
    wfeat, bfeat = p['feat_emb']
    wfa_t = wanf[:, 0:A].T
    wfb_t = wanf[:, A:2 * A].T
    banf_r = banf.reshape(1, A)
    wpa_t = wpos[:, 0:128].T[:, perm]
    wpb_t = wpos[:, 128:640].T[:, perm]
    bpos_r = bpos[perm].reshape(1, 512)
    wfeat_t = wfeat.T
    bfeat_r = bfeat.reshape(1, 512)
    bng0 = p['phi_pos']['bn_g'][perm]
    bnb0 = p['phi_pos']['bn_b'][perm]

    w1f = p['c1'][0].transpose(2, 1, 0).reshape(192, 512)
    b1c = p['c1'][1].reshape(1, 512)
    w2f = p['c2'][0].transpose(2, 1, 0).reshape(1536, 512)
    b2c = p['c2'][1].reshape(1, 512)
    w3f = p['c3'][0].transpose(2, 1, 0).reshape(1536, 256)
    b3c = p['c3'][1].reshape(1, 256)
    w4f = p['c4'][0].transpose(2, 1, 0).reshape(768, 256)
    b4c = p['c4'][1].reshape(1, 256)
    w5p = jnp.zeros((1024, 8), f32).at[:, 0].set(
        p['c5'][0].transpose(2, 1, 0).reshape(1024))
    b5p = jnp.zeros((1, 8), f32).at[0, 0].set(p['c5'][1][0])

    c0 = convs[0]
    x, xa, xb = _emb_pre(atom_fea, wn_t, bn_, c0['wa_t'], c0['wb_t'])
    src = nbr_fea
    e = None
    for li, c in enumerate(convs):
        g1, g2 = _sc_gather_pair(xa, xb, idx1, idx2)
        ew = [c['w1c_t'], c['b1'], c['w2_t'], c['b2'], c['w3_t'], c['b3']]
        if li == 0:
            ew = [we_t, be_] + ew
        ek, e = _edge_mlp(src, g1, g2, ew, has_emb=(li == 0))
        rho_p = _sc_scatter_add(ek, idx1, zeros)
        wv = [c['wv1a_t'], c['bv1'], c['wv1b_t'], c['wv2_t'], c['bv2'],
              c['wv3_t'], c['bv3'], c['bng'], c['bnb']]
        x = _node_mlp(x, rho_p, invn, wv)
        if li < 2:
            nc = convs[li + 1]
            xa, xb = _node_pre(x, nc['wa_t'], nc['wb_t'])
        src = e

    eks_p = _sc_scatter_add(e, idx1, zeros)

    sf2 = _sf_table(structure_feature, wfeat_t, bfeat_r)
    sf2g = _sc_gather_sf(sf2, ai)
    z, st0 = _charge_pre(x, eks_p, invn, sf2g, wfa_t, wfb_t, banf_r,
                         wpa_t, wpb_t, bpos_r)
    sc0, of0 = _bn_affine(z, st0[0], float(N), bng0, bnb0)
    h1, st1 = _conv1(z, sc0.reshape(1, 512), of0.reshape(1, 512), w1f, b1c)
    sc1, of1 = _bn_affine(h1, st1[0], float(N * 6), p['bn1'][0], p['bn1'][1])
    h2, st2 = _conv2(h1, jnp.tile(sc1, 6).reshape(1, 3072),
                     jnp.tile(of1, 6).reshape(1, 3072), w2f, b2c)
    sc2, of2 = _bn_affine(h2, st2[0], float(N * 4), p['bn2'][0], p['bn2'][1])
    res = _conv345(h2, jnp.tile(sc2, 4).reshape(1, 2048),
                   jnp.tile(of2, 4).reshape(1, 2048),
                   w3f, b3c, w4f, b4c, w5p, b5p)
    return res[:, 0]


# speedup vs baseline: 1.2513x; 1.2513x over previous
import functools

import jax
import jax.numpy as jnp
from jax import lax
from jax.experimental import pallas as pl
from jax.experimental.pallas import tpu as pltpu
from jax.experimental.pallas import tpu_sc as plsc

N = 10000
E = 320000
A = 128
NCRYS = 200
NW = 32
PER_W = E // NW
CH = 80
NCHUNK = PER_W // CH
NR_CH = 40
NR_NCHUNK = N // NR_CH

def _lrelu(x):
    return jnp.where(x >= 0, x, 0.2 * x)


def _sc_mesh():
    return plsc.VectorSubcoreMesh(core_axis_name="c", subcore_axis_name="s")


GCH = 128
NFULL = PER_W // GCH
GTAIL = PER_W - NFULL * GCH


def _sc_gather_pair_body(xa_hbm, xb_hbm, i1_hbm, i2_hbm, g1_hbm, g2_hbm,
                         i1v, i2v, r1a, r1b, r1c, r2a, r2b, r2c,
                         gs0, gs1, gs2, ws0, ws1, ws2):
    wid = lax.axis_index("s") * 2 + lax.axis_index("c")
    base = wid * PER_W
    r1 = (r1a, r1b, r1c)
    r2 = (r2a, r2b, r2c)
    gs = (gs0, gs1, gs2)
    ws = (ws0, ws1, ws2)
    pltpu.sync_copy(i1_hbm.at[pl.ds(base, PER_W)], i1v)
    pltpu.sync_copy(i2_hbm.at[pl.ds(base, PER_W)], i2v)

    def g_descs(c, b):
        s = pl.ds(c * GCH, GCH)
        return (pltpu.make_async_copy(xa_hbm.at[i1v.at[s]], r1[b], gs[b]),
                pltpu.make_async_copy(xb_hbm.at[i2v.at[s]], r2[b], gs[b]))

    def w_descs(c, b):
        off = base + c * GCH
        return (pltpu.make_async_copy(r1[b], g1_hbm.at[pl.ds(off, GCH)], ws[b]),
                pltpu.make_async_copy(r2[b], g2_hbm.at[pl.ds(off, GCH)], ws[b]))

    def issue(descs):
        for d_ in descs:
            d_.start()

    def drain(descs):
        for d_ in descs:
            d_.wait()

    issue(g_descs(0, 0))
    issue(g_descs(1, 1))

    def group(gi, carry):
        g0 = gi * 3
        for u in range(3):
            g = g0 + u
            b = u
            nb = (u + 2) % 3
            drain(g_descs(g, b))
            issue(w_descs(g, b))

            @pl.when(g >= 1)
            def _():
                drain(w_descs(g - 1, nb))

            @pl.when(g + 2 < NFULL)
            def _():
                issue(g_descs(g + 2, nb))

        return carry

    lax.fori_loop(0, NFULL // 3, group, 0)
    drain(w_descs(NFULL - 1, (NFULL - 1) % 3))
    ts = pl.ds(NFULL * GCH, GTAIL)
    toff = base + NFULL * GCH
    pltpu.async_copy(xa_hbm.at[i1v.at[ts]], r1a.at[pl.ds(0, GTAIL)], gs0).wait()
    pltpu.async_copy(xb_hbm.at[i2v.at[ts]], r2a.at[pl.ds(0, GTAIL)], gs0).wait()
    pltpu.sync_copy(r1a.at[pl.ds(0, GTAIL)], g1_hbm.at[pl.ds(toff, GTAIL)])
    pltpu.sync_copy(r2a.at[pl.ds(0, GTAIL)], g2_hbm.at[pl.ds(toff, GTAIL)])


@functools.cache
def _sc_gather_pair_call():
    return pl.kernel(
        _sc_gather_pair_body,
        out_type=(jax.ShapeDtypeStruct((E, A), jnp.float32),
                  jax.ShapeDtypeStruct((E, A), jnp.float32)),
        mesh=_sc_mesh(),
        scratch_types=[
            pltpu.VMEM((PER_W,), jnp.int32),
            pltpu.VMEM((PER_W,), jnp.int32),
            pltpu.VMEM((GCH, A), jnp.float32),
            pltpu.VMEM((GCH, A), jnp.float32),
            pltpu.VMEM((GCH, A), jnp.float32),
            pltpu.VMEM((GCH, A), jnp.float32),
            pltpu.VMEM((GCH, A), jnp.float32),
            pltpu.VMEM((GCH, A), jnp.float32),
            pltpu.SemaphoreType.DMA,
            pltpu.SemaphoreType.DMA,
            pltpu.SemaphoreType.DMA,
            pltpu.SemaphoreType.DMA,
            pltpu.SemaphoreType.DMA,
            pltpu.SemaphoreType.DMA,
        ],
    )


def _sc_gather_pair(xa, xb, i1, i2):
    return _sc_gather_pair_call()(xa, xb, i1, i2)


EHALF = E // 2
HPER_W = EHALF // NW
HCH = 40
HNCHUNK = HPER_W // HCH


def _sc_scatter_add_body(half, rows_hbm, i1_hbm, zeros_hbm, out_hbm,
                         iva, ivb, ivc, ra, rb, rc, fs0, fs1, fs2, acc):
    cid = lax.axis_index("c")
    sid = lax.axis_index("s")
    wid = sid * 2 + cid
    base = half * EHALF + wid * HPER_W
    iv = (iva, ivb, ivc)
    r = (ra, rb, rc)
    fs = (fs0, fs1, fs2)
    init_iters = (NR_NCHUNK + 15) // 16

    def init(t, carry):
        c = sid + t * 16

        @pl.when(c < NR_NCHUNK)
        def _():
            pltpu.sync_copy(zeros_hbm.at[pl.ds(c * NR_CH, NR_CH)], ra)
            pltpu.sync_copy(ra, acc.at[pl.ds(c * NR_CH, NR_CH)])

        return carry

    lax.fori_loop(0, init_iters, init, 0)
    plsc.subcore_barrier()

    def f_descs(c, b):
        off = base + c * HCH
        return (pltpu.make_async_copy(i1_hbm.at[pl.ds(off, HCH)], iv[b], fs[b]),
                pltpu.make_async_copy(rows_hbm.at[pl.ds(off, HCH)], r[b], fs[b]))

    for d_ in f_descs(0, 0) + f_descs(1, 1):
        d_.start()

    def group(gi, carry):
        g0 = gi * 3
        for u in range(3):
            g = g0 + u
            b = u
            nb = (u + 2) % 3

            @pl.when(g < HNCHUNK)
            def _():
                for d_ in f_descs(g, b):
                    d_.wait()
                pltpu.sync_copy(r[b], acc.at[iv[b]], add=True)

            @pl.when(g + 2 < HNCHUNK)
            def _():
                for d_ in f_descs(g + 2, nb):
                    d_.start()

        return carry

    lax.fori_loop(0, (HNCHUNK + 2) // 3, group, 0)
    plsc.subcore_barrier()

    def drain(t, carry):
        c = sid + t * 16

        @pl.when(c < NR_NCHUNK)
        def _():
            pltpu.sync_copy(acc.at[pl.ds(c * NR_CH, NR_CH)], ra)
            pltpu.sync_copy(ra, out_hbm.at[cid, pl.ds(c * NR_CH, NR_CH)])

        return carry

    lax.fori_loop(0, init_iters, drain, 0)


@functools.cache
def _sc_scatter_add_call(half):
    return pl.kernel(
        functools.partial(_sc_scatter_add_body, half),
        out_type=jax.ShapeDtypeStruct((2, N, A), jnp.float32),
        mesh=_sc_mesh(),
        scratch_types=[
            pltpu.VMEM((HCH,), jnp.int32),
            pltpu.VMEM((HCH,), jnp.int32),
            pltpu.VMEM((HCH,), jnp.int32),
            pltpu.VMEM((HCH, A), jnp.float32),
            pltpu.VMEM((HCH, A), jnp.float32),
            pltpu.VMEM((HCH, A), jnp.float32),
            pltpu.SemaphoreType.DMA,
            pltpu.SemaphoreType.DMA,
            pltpu.SemaphoreType.DMA,
            pltpu.VMEM_SHARED((N, A), jnp.float32),
        ],
    )


def _sc_scatter_add(rows, i1, zeros):
    return (_sc_scatter_add_call(0)(rows, i1, zeros),
            _sc_scatter_add_call(1)(rows, i1, zeros))


_SF_CH = 80
_SF_NCHUNK = N // _SF_CH


def _sc_gather_sf_body(tab_hbm, ai_hbm, out_hbm, iv, rows, sem):
    wid = lax.axis_index("s") * 2 + lax.axis_index("c")
    iters = (_SF_NCHUNK + NW - 1) // NW

    def body(t, carry):
        c = wid + t * NW

        @pl.when(c < _SF_NCHUNK)
        def _():
            off = c * _SF_CH
            pltpu.sync_copy(ai_hbm.at[pl.ds(off, _SF_CH)], iv)
            pltpu.async_copy(tab_hbm.at[iv], rows, sem).wait()
            pltpu.sync_copy(rows, out_hbm.at[pl.ds(off, _SF_CH)])

        return carry

    lax.fori_loop(0, iters, body, 0)


@functools.cache
def _sc_gather_sf_call():
    return pl.kernel(
        _sc_gather_sf_body,
        out_type=jax.ShapeDtypeStruct((N, 512), jnp.float32),
        mesh=_sc_mesh(),
        scratch_types=[
            pltpu.VMEM((_SF_CH,), jnp.int32),
            pltpu.VMEM((_SF_CH, 512), jnp.float32),
            pltpu.SemaphoreType.DMA,
        ],
    )


def _sc_gather_sf(tab, ai):
    return _sc_gather_sf_call()(tab, ai)


def _dot(a, b):
    return jnp.dot(a, b, preferred_element_type=jnp.float32)


def _emb_pre_body(af, wn, bn, wa, wb, x_o, xa_o, xb_o):
    x = _dot(af[...], wn[...]) + bn[...]
    x_o[...] = x
    xa_o[...] = _dot(x, wa[...])
    xb_o[...] = _dot(x, wb[...])


def _emb_pre(atom_fea, wn_t, bn, wa_t, wb_t):
    f = jax.ShapeDtypeStruct
    return pl.pallas_call(
        _emb_pre_body,
        out_shape=(f((N, A), jnp.float32), f((N, A), jnp.float32),
                   f((N, A), jnp.float32)),
    )(atom_fea, wn_t, bn, wa_t, wb_t)


EB = 1600
EGRID = E // EB


def _edge_mlp_body(has_emb, src, g1, g2, *refs):
    if has_emb:
        we, be = refs[0], refs[1]
        refs = refs[2:]
    w1c, b1, w2, b2, w3, b3, ek_o, enew_o = refs
    if has_emb:
        e = _dot(src[...], we[...]) + be[...]
    else:
        e = src[...]
    t = _lrelu(_dot(e, w1c[...]) + g1[...] + g2[...] + b1[...])
    t = _lrelu(_dot(t, w2[...]) + b2[...])
    ek = _dot(t, w3[...]) + b3[...]
    ek_o[...] = ek
    enew_o[...] = e + ek


def _edge_mlp(src, g1, g2, weights, has_emb):
    f = jax.ShapeDtypeStruct
    kin = src.shape[1]
    full = lambda s: pl.BlockSpec(s, lambda i: (0, 0))
    row = lambda s: pl.BlockSpec(s, lambda i: (i, 0))
    in_specs = [row((EB, kin)), row((EB, A)), row((EB, A))]
    in_specs += [full(w.shape) for w in weights]
    return pl.pallas_call(
        functools.partial(_edge_mlp_body, has_emb),
        grid=(EGRID,),
        in_specs=in_specs,
        out_specs=(row((EB, A)), row((EB, A))),
        out_shape=(f((E, A), jnp.float32), f((E, A), jnp.float32)),
    )(src, g1, g2, *weights)


def _node_mlp_body(x, rho_a, rho_b, invn, wv1a, bv1, wv1b, wv2, bv2, wv3, bv3,
                   bng, bnb, xn_o):
    rho = ((rho_a[0] + rho_a[1]) + (rho_b[0] + rho_b[1])) * invn[...]
    t = _lrelu(_dot(x[...], wv1a[...]) + _dot(rho, wv1b[...]) + bv1[...])
    t = _lrelu(_dot(t, wv2[...]) + bv2[...])
    vi = _dot(t, wv3[...]) + bv3[...]
    m = jnp.mean(vi, axis=0, keepdims=True)
    d = vi - m
    v = jnp.mean(d * d, axis=0, keepdims=True)
    vi = bng[...] * d / jnp.sqrt(v + 1e-5) + bnb[...]
    xn_o[...] = x[...] + vi


def _node_mlp(x, rho_p, invn, wv):
    return pl.pallas_call(
        _node_mlp_body,
        out_shape=jax.ShapeDtypeStruct((N, A), jnp.float32),
    )(x, rho_p[0], rho_p[1], invn, *wv)


NPB = 2000


def _node_pre_body(x, wa, wb, xa_o, xb_o):
    xa_o[...] = _dot(x[...], wa[...])
    xb_o[...] = _dot(x[...], wb[...])


def _node_pre(x, wa_t, wb_t):
    f = jax.ShapeDtypeStruct
    full = lambda s: pl.BlockSpec(s, lambda i: (0, 0))
    row = lambda s: pl.BlockSpec(s, lambda i: (i, 0))
    return pl.pallas_call(
        _node_pre_body,
        grid=(N // NPB,),
        in_specs=[row((NPB, A)), full((A, A)), full((A, A))],
        out_specs=(row((NPB, A)), row((NPB, A))),
        out_shape=(f((N, A), jnp.float32), f((N, A), jnp.float32)),
    )(x, wa_t, wb_t)


def _sf_table_body(sf, wf, bf, o):
    o[...] = _dot(sf[...], wf[...]) + bf[...]


def _sf_table(structure_feature, wfeat_t, bfeat):
    return pl.pallas_call(
        _sf_table_body,
        out_shape=jax.ShapeDtypeStruct((NCRYS, 512), jnp.float32),
    )(structure_feature, wfeat_t, bfeat)


ZB = 2000


def _charge_pre_body(x, eks_a, eks_b, invn, sf2g, wfa, wfb, banf, wpa, wpb,
                     bpos, z_o, st_o):
    eks = ((eks_a[0] + eks_a[1]) + (eks_b[0] + eks_b[1])) * invn[...]
    anf_lin = _dot(x[...], wfa[...]) + _dot(eks, wfb[...]) + banf[...]
    z = (_dot(anf_lin, wpa[...]) + _dot(sf2g[...], wpb[...]) + bpos[...])
    z_o[...] = z

    @pl.when(pl.program_id(0) == 0)
    def _():
        st_o[...] = jnp.zeros_like(st_o)

    st_o[...] += jnp.sum(z, axis=0, keepdims=True)


def _charge_pre(x, eks_p, invn, sf2g, wfa_t, wfb_t, banf, wpa_t, wpb_t, bpos):
    f = jax.ShapeDtypeStruct
    full = lambda s: pl.BlockSpec(s, lambda i: tuple(0 for _ in s))
    row = lambda s: pl.BlockSpec(s, lambda i: (i,) + tuple(0 for _ in s[1:]))
    return pl.pallas_call(
        _charge_pre_body,
        grid=(N // ZB,),
        in_specs=[row((ZB, A)), pl.BlockSpec((2, ZB, A), lambda i: (0, i, 0)),
                  pl.BlockSpec((2, ZB, A), lambda i: (0, i, 0)),
                  row((ZB, 1)), row((ZB, 512)),
                  full((A, A)), full((A, A)), full((1, A)),
                  full((A, 512)), full((512, 512)), full((1, 512))],
        out_specs=(row((ZB, 512)), full((1, 512))),
        out_shape=(f((N, 512), jnp.float32), f((1, 512), jnp.float32)),
    )(x, eks_p[0], eks_p[1], invn, sf2g, wfa_t, wfb_t, banf, wpa_t, wpb_t,
      bpos)


VB = 2000


def _colvar_body(nslice, a, m, st_o):
    @pl.when(pl.program_id(0) == 0)
    def _():
        st_o[...] = jnp.zeros_like(st_o)

    c = m.shape[1]
    acc = jnp.zeros((1, c), jnp.float32)
    for p in range(nslice):
        d = a[:, p * c:(p + 1) * c] - m[...]
        acc += jnp.sum(d * d, axis=0, keepdims=True)
    st_o[...] += acc


def _colvar(a, mean):
    nslice = a.shape[1] // mean.shape[1]
    c = mean.shape[1]
    full = lambda s: pl.BlockSpec(s, lambda i: (0, 0))
    row = lambda s: pl.BlockSpec(s, lambda i: (i, 0))
    return pl.pallas_call(
        functools.partial(_colvar_body, nslice),
        grid=(N // VB,),
        in_specs=[row((VB, a.shape[1])), full((1, c))],
        out_specs=full((1, c)),
        out_shape=jax.ShapeDtypeStruct((1, c), jnp.float32),
    )(a, mean)


CB1 = 1000
CB2 = 400


def _conv1_body(z, scale, offs, w1f, b1c, h1_o, st_o):
    c = _lrelu(z[...] * scale[...] + offs[...])

    @pl.when(pl.program_id(0) == 0)
    def _():
        st_o[...] = jnp.zeros_like(st_o)

    s = jnp.zeros((1, 512), jnp.float32)
    for p in range(6):
        hp = _dot(c[:, p * 64:(p + 3) * 64], w1f[...]) + b1c[...]
        h1_o[:, p * 512:(p + 1) * 512] = hp
        s += jnp.sum(hp, axis=0, keepdims=True)
    st_o[...] += s


def _conv1(z, scale, offs, w1f, b1c):
    f = jax.ShapeDtypeStruct
    full = lambda s: pl.BlockSpec(s, lambda i: tuple(0 for _ in s))
    row = lambda s: pl.BlockSpec(s, lambda i: (i,) + tuple(0 for _ in s[1:]))
    return pl.pallas_call(
        _conv1_body,
        grid=(N // CB1,),
        in_specs=[row((CB1, 512)), full((1, 512)), full((1, 512)),
                  full((192, 512)), full((1, 512))],
        out_specs=(row((CB1, 3072)), full((1, 512))),
        out_shape=(f((N, 3072), jnp.float32), f((1, 512), jnp.float32)),
    )(z, scale, offs, w1f, b1c)


def _conv2_body(h1, scale, offs, w2f, b2c, h2_o, st_o):
    a = _lrelu(h1[...] * scale[...] + offs[...])

    @pl.when(pl.program_id(0) == 0)
    def _():
        st_o[...] = jnp.zeros_like(st_o)

    s = jnp.zeros((1, 512), jnp.float32)
    for q in range(4):
        hq = _dot(a[:, q * 512:(q + 3) * 512], w2f[...]) + b2c[...]
        h2_o[:, q * 512:(q + 1) * 512] = hq
        s += jnp.sum(hq, axis=0, keepdims=True)
    st_o[...] += s


def _conv2(h1, scale_t, offs_t, w2f, b2c):
    f = jax.ShapeDtypeStruct
    full = lambda s: pl.BlockSpec(s, lambda i: tuple(0 for _ in s))
    row = lambda s: pl.BlockSpec(s, lambda i: (i,) + tuple(0 for _ in s[1:]))
    return pl.pallas_call(
        _conv2_body,
        grid=(N // CB2,),
        in_specs=[row((CB2, 3072)), full((1, 3072)), full((1, 3072)),
                  full((1536, 512)), full((1, 512))],
        out_specs=(row((CB2, 2048)), full((1, 512))),
        out_shape=(f((N, 2048), jnp.float32), f((1, 512), jnp.float32)),
    )(h1, scale_t, offs_t, w2f, b2c)


def _conv345_body(h2, scale, offs, w3f, b3c, w4f, b4c, w5p, b5p, out_o):
    a = _lrelu(h2[...] * scale[...] + offs[...])
    h3 = []
    for q in range(4):
        if q == 0:
            hq = _dot(a[:, 0:1024], w3f[512:1536, :])
        elif q == 3:
            hq = _dot(a[:, 1024:2048], w3f[0:1024, :])
        else:
            hq = _dot(a[:, (q - 1) * 512:(q + 2) * 512], w3f[...])
        h3.append(_lrelu(hq + b3c[...]))
    h3 = jnp.concatenate(h3, axis=1)
    h4 = []
    for q in range(4):
        if q == 0:
            hq = _dot(h3[:, 0:512], w4f[256:768, :])
        elif q == 3:
            hq = _dot(h3[:, 512:1024], w4f[0:512, :])
        else:
            hq = _dot(h3[:, (q - 1) * 256:(q + 2) * 256], w4f[...])
        h4.append(_lrelu(hq + b4c[...]))
    h4 = jnp.concatenate(h4, axis=1)
    out_o[...] = _dot(h4, w5p[...]) + b5p[...]


def _conv345(h2, scale_t, offs_t, w3f, b3c, w4f, b4c, w5p, b5p):
    full = lambda s: pl.BlockSpec(s, lambda i: tuple(0 for _ in s))
    row = lambda s: pl.BlockSpec(s, lambda i: (i,) + tuple(0 for _ in s[1:]))
    return pl.pallas_call(
        _conv345_body,
        grid=(N // CB2,),
        in_specs=[row((CB2, 2048)), full((1, 2048)), full((1, 2048)),
                  full((1536, 256)), full((1, 256)),
                  full((768, 256)), full((1, 256)),
                  full((1024, 8)), full((1, 8))],
        out_specs=row((CB2, 8)),
        out_shape=jax.ShapeDtypeStruct((N, 8), jnp.float32),
    )(h2, scale_t, offs_t, w3f, b3c, w4f, b4c, w5p, b5p)


def _bn_affine(a, ssum, count, g, b):
    m = ssum / count
    ss = _colvar(a, m.reshape(1, -1))
    v = ss[0] / count
    scale = g / jnp.sqrt(v + 1e-5)
    return scale, b - m * scale


def kernel(atom_fea, nbr_fea, nbr_fea_idx1, nbr_fea_idx2, num_nbrs, atom_idx,
           structure_feature, params):
    p = params
    f32 = jnp.float32
    idx1 = nbr_fea_idx1.astype(jnp.int32)
    idx2 = nbr_fea_idx2.astype(jnp.int32)
    ai = atom_idx.astype(jnp.int32)
    invn = (1.0 / num_nbrs).reshape(N, 1).astype(f32)
    zeros = jnp.zeros((N, A), f32)

    wn_t = p['node_emb'][0].T
    bn_ = p['node_emb'][1].reshape(1, A)
    we_t = p['edge_emb'][0].T
    be_ = p['edge_emb'][1].reshape(1, A)

    convs = []
    for c in p['convs']:
        w1 = c['phi_e'][0][0]
        convs.append(dict(
            wa_t=w1[:, 0:A].T, wb_t=w1[:, A:2 * A].T, w1c_t=w1[:, 2 * A:].T,
            b1=c['phi_e'][0][1].reshape(1, A),
            w2_t=c['phi_e'][1][0].T, b2=c['phi_e'][1][1].reshape(1, A),
            w3_t=c['phi_e'][2][0].T, b3=c['phi_e'][2][1].reshape(1, A),
            wv1a_t=c['phi_v'][0][0][:, 0:A].T,
            wv1b_t=c['phi_v'][0][0][:, A:2 * A].T,
            bv1=c['phi_v'][0][1].reshape(1, A),
            wv2_t=c['phi_v'][1][0].T, bv2=c['phi_v'][1][1].reshape(1, A),
            wv3_t=c['phi_v'][2][0].T, bv3=c['phi_v'][2][1].reshape(1, A),
            bng=c['bn_g'].reshape(1, A), bnb=c['bn_b'].reshape(1, A),
        ))

    hh, cc = jnp.meshgrid(jnp.arange(8), jnp.arange(64), indexing='ij')
    perm = (cc * 8 + hh).reshape(512)
    wanf, banf = p['anf_emb']
    wpos, bpos = p['phi_pos']['lin']
    wfeat, bfeat = p['feat_emb']
    wfa_t = wanf[:, 0:A].T
    wfb_t = wanf[:, A:2 * A].T
    banf_r = banf.reshape(1, A)
    wpa_t = wpos[:, 0:128].T[:, perm]
    wpb_t = wpos[:, 128:640].T[:, perm]
    bpos_r = bpos[perm].reshape(1, 512)
    wfeat_t = wfeat.T
    bfeat_r = bfeat.reshape(1, 512)
    bng0 = p['phi_pos']['bn_g'][perm]
    bnb0 = p['phi_pos']['bn_b'][perm]

    w1f = p['c1'][0].transpose(2, 1, 0).reshape(192, 512)
    b1c = p['c1'][1].reshape(1, 512)
    w2f = p['c2'][0].transpose(2, 1, 0).reshape(1536, 512)
    b2c = p['c2'][1].reshape(1, 512)
    w3f = p['c3'][0].transpose(2, 1, 0).reshape(1536, 256)
    b3c = p['c3'][1].reshape(1, 256)
    w4f = p['c4'][0].transpose(2, 1, 0).reshape(768, 256)
    b4c = p['c4'][1].reshape(1, 256)
    w5p = jnp.zeros((1024, 8), f32).at[:, 0].set(
        p['c5'][0].transpose(2, 1, 0).reshape(1024))
    b5p = jnp.zeros((1, 8), f32).at[0, 0].set(p['c5'][1][0])

    c0 = convs[0]
    x, xa, xb = _emb_pre(atom_fea, wn_t, bn_, c0['wa_t'], c0['wb_t'])
    src = nbr_fea
    e = None
    for li, c in enumerate(convs):
        g1, g2 = _sc_gather_pair(xa, xb, idx1, idx2)
        ew = [c['w1c_t'], c['b1'], c['w2_t'], c['b2'], c['w3_t'], c['b3']]
        if li == 0:
            ew = [we_t, be_] + ew
        ek, e = _edge_mlp(src, g1, g2, ew, has_emb=(li == 0))
        rho_p = _sc_scatter_add(ek, idx1, zeros)
        wv = [c['wv1a_t'], c['bv1'], c['wv1b_t'], c['wv2_t'], c['bv2'],
              c['wv3_t'], c['bv3'], c['bng'], c['bnb']]
        x = _node_mlp(x, rho_p, invn, wv)
        if li < 2:
            nc = convs[li + 1]
            xa, xb = _node_pre(x, nc['wa_t'], nc['wb_t'])
        src = e

    eks_p = _sc_scatter_add(e, idx1, zeros)

    sf2 = _sf_table(structure_feature, wfeat_t, bfeat_r)
    sf2g = _sc_gather_sf(sf2, ai)
    z, st0 = _charge_pre(x, eks_p, invn, sf2g, wfa_t, wfb_t, banf_r,
                         wpa_t, wpb_t, bpos_r)
    sc0, of0 = _bn_affine(z, st0[0], float(N), bng0, bnb0)
    h1, st1 = _conv1(z, sc0.reshape(1, 512), of0.reshape(1, 512), w1f, b1c)
    sc1, of1 = _bn_affine(h1, st1[0], float(N * 6), p['bn1'][0], p['bn1'][1])
    h2, st2 = _conv2(h1, jnp.tile(sc1, 6).reshape(1, 3072),
                     jnp.tile(of1, 6).reshape(1, 3072), w2f, b2c)
    sc2, of2 = _bn_affine(h2, st2[0], float(N * 4), p['bn2'][0], p['bn2'][1])
    res = _conv345(h2, jnp.tile(sc2, 4).reshape(1, 2048),
                   jnp.tile(of2, 4).reshape(1, 2048),
                   w3f, b3c, w4f, b4c, w5p, b5p)
    return res[:, 0]
